# baseline (speedup 1.0000x reference)
import functools

import jax
import jax.numpy as jnp
from jax import lax
from jax.experimental import pallas as pl
from jax.experimental.pallas import tpu as pltpu
from jax.experimental.pallas import tpu_sc as plsc

VOCAB = 100000
BATCH = 16384
LANES = 16
NW = 32
V_PER_W = VOCAB // NW
V_PAD = 3136
CE_PAD = 3 * V_PAD
N_VECS = BATCH // LANES
HIST_WORDS = LANES * V_PAD

ROWS = 200
COLS = 1500
GRID_R = 5
BR = ROWS // GRID_R

_mesh = plsc.VectorSubcoreMesh(core_axis_name="c", subcore_axis_name="s")


@functools.partial(
    pl.kernel,
    mesh=_mesh,
    out_type=jax.ShapeDtypeStruct((NW, CE_PAD), jnp.float32),
    scratch_types=[
        pltpu.VMEM((BATCH,), jnp.int32),
        pltpu.VMEM((HIST_WORDS,), jnp.float32),
        pltpu.VMEM((CE_PAD,), jnp.float32),
    ],
    compiler_params=pltpu.CompilerParams(needs_layout_passes=False),
)
def _sc_expanded_histogram(idx_hbm, out_hbm, idx_v, hist_v, ce_v):
    wid = lax.axis_index("s") * 2 + lax.axis_index("c")
    vbase = wid * V_PER_W
    pltpu.sync_copy(idx_hbm, idx_v)

    zeros = jnp.zeros((LANES,), jnp.float32)

    def _zero(i, carry):
        hist_v[pl.ds(i * LANES, LANES)] = zeros
        return carry

    lax.fori_loop(0, HIST_WORDS // LANES, _zero, 0)

    lane = lax.iota(jnp.int32, LANES)
    ones = jnp.ones((LANES,), jnp.float32)

    def _scatter(k, carry):
        idx16 = idx_v[pl.ds(k * LANES, LANES)]
        loc = idx16 - vbase
        m = (loc >= 0) & (loc < V_PER_W)
        tgt = lane * V_PAD + loc
        plsc.addupdate_scatter(hist_v, [tgt], ones, mask=m)
        return carry

    lax.fori_loop(0, N_VECS, _scatter, 0)

    def _reduce(c, carry):
        base = c * LANES
        s = hist_v[pl.ds(base, LANES)]
        for L in range(1, LANES):
            s = s + hist_v[pl.ds(L * V_PAD + base, LANES)]
        t3 = (base + lane) * 3
        plsc.store_scatter(ce_v, [t3], s)
        plsc.store_scatter(ce_v, [t3 + 1], s)
        plsc.store_scatter(ce_v, [t3 + 2], s)
        return carry

    lax.fori_loop(0, V_PAD // LANES, _reduce, 0)

    pltpu.sync_copy(ce_v, out_hbm.at[wid])


def _tc_body(ce_ref, t0_ref, t1_ref, t2_ref, out_ref, acc_ref):
    r = pl.program_id(0)

    @pl.when(r == 0)
    def _():
        acc_ref[...] = jnp.zeros((3, COLS), jnp.float32)

    ce = ce_ref[...]
    for g, tref, ntab in ((0, t0_ref, 5), (1, t1_ref, 10), (2, t2_ref, 6)):
        s = tref[0]
        for t in range(1, ntab):
            s = s + tref[t]
        acc_ref[g:g + 1, :] += jnp.sum(ce * s, axis=0, keepdims=True)

    @pl.when(r == GRID_R - 1)
    def _():
        j = lax.broadcasted_iota(jnp.int32, (COLS, 3), 0)
        d = lax.broadcasted_iota(jnp.int32, (COLS, 3), 1)
        sel = (j % 3 == d).astype(jnp.float32)
        out_ref[...] = jnp.dot(acc_ref[...], sel,
                               preferred_element_type=jnp.float32)


def _tc_reduce(ce, t0, t1, t2):
    return pl.pallas_call(
        _tc_body,
        grid=(GRID_R,),
        in_specs=[
            pl.BlockSpec((BR, COLS), lambda r: (r, 0)),
            pl.BlockSpec((5, BR, COLS), lambda r: (0, r, 0)),
            pl.BlockSpec((10, BR, COLS), lambda r: (0, r, 0)),
            pl.BlockSpec((6, BR, COLS), lambda r: (0, r, 0)),
        ],
        out_specs=pl.BlockSpec((3, 3), lambda r: (0, 0)),
        out_shape=jax.ShapeDtypeStruct((3, 3), jnp.float32),
        scratch_shapes=[pltpu.VMEM((3, COLS), jnp.float32)],
    )(ce, t0, t1, t2)


def kernel(e_input, tables0, tables1, tables2):
    idx = e_input.astype(jnp.int32)
    ce_pad = _sc_expanded_histogram(idx)
    ce = ce_pad[:, :3 * V_PER_W].reshape(ROWS, COLS)
    t0 = tables0.reshape(5, ROWS, COLS)
    t1 = tables1.reshape(10, ROWS, COLS)
    t2 = tables2.reshape(6, ROWS, COLS)
    return _tc_reduce(ce, t0, t1, t2)


# speedup vs baseline: 47.1320x; 47.1320x over previous
import functools

import jax
import jax.numpy as jnp
from jax import lax
from jax.experimental import pallas as pl
from jax.experimental.pallas import tpu as pltpu
from jax.experimental.pallas import tpu_sc as plsc

VOCAB = 100000
BATCH = 16384
LANES = 16
NW = 32
V_PER_W = 3200
N_VECS = BATCH // LANES
HIST_WORDS = LANES * V_PER_W

VB = 4096
GRID_B = 25
R0, R1, R2 = 15, 30, 18

_mesh = plsc.VectorSubcoreMesh(core_axis_name="c", subcore_axis_name="s")


@functools.partial(
    pl.kernel,
    mesh=_mesh,
    out_type=jax.ShapeDtypeStruct((VOCAB,), jnp.float32),
    scratch_types=[
        pltpu.VMEM((BATCH,), jnp.int32),
        pltpu.VMEM((HIST_WORDS,), jnp.float32),
        pltpu.VMEM((V_PER_W,), jnp.float32),
    ],
    compiler_params=pltpu.CompilerParams(needs_layout_passes=False),
)
def _sc_histogram(idx_hbm, out_hbm, idx_v, hist_v, cnt_v):
    wid = lax.axis_index("s") * 2 + lax.axis_index("c")
    vbase = wid * V_PER_W
    pltpu.sync_copy(idx_hbm, idx_v)

    zeros = jnp.zeros((LANES,), jnp.float32)

    def _zero(i, carry):
        hist_v[pl.ds(i * LANES, LANES)] = zeros
        return carry

    lax.fori_loop(0, HIST_WORDS // LANES, _zero, 0)

    lane = lax.iota(jnp.int32, LANES)
    ones = jnp.ones((LANES,), jnp.float32)

    def _scatter(k, carry):
        idx16 = idx_v[pl.ds(k * LANES, LANES)]
        loc = idx16 - vbase
        m = (loc >= 0) & (loc < V_PER_W)
        tgt = lane * V_PER_W + loc
        plsc.addupdate_scatter(hist_v, [tgt], ones, mask=m)
        return carry

    lax.fori_loop(0, N_VECS, _scatter, 0)

    def _reduce(c, carry):
        base = c * LANES
        s = hist_v[pl.ds(base, LANES)]
        for ln in range(1, LANES):
            s = s + hist_v[pl.ds(ln * V_PER_W + base, LANES)]
        cnt_v[pl.ds(base, LANES)] = s
        return carry

    lax.fori_loop(0, V_PER_W // LANES, _reduce, 0)

    n_out = VOCAB - (NW - 1) * V_PER_W

    @pl.when(wid < NW - 1)
    def _():
        pltpu.sync_copy(cnt_v, out_hbm.at[pl.ds(vbase, V_PER_W)])

    @pl.when(wid == NW - 1)
    def _():
        pltpu.sync_copy(cnt_v.at[pl.ds(0, n_out)],
                        out_hbm.at[pl.ds((NW - 1) * V_PER_W, n_out)])


def _tc_body(cnt_ref, x0_ref, x1_ref, x2_ref, out_ref, acc_ref):
    b = pl.program_id(0)

    @pl.when(b == 0)
    def _():
        acc_ref[...] = jnp.zeros((R0 + R1 + R2, VB), jnp.float32)

    v = b * VB + lax.broadcasted_iota(jnp.int32, (1, VB), 1)
    valid = v < VOCAB
    cm = cnt_ref[...]
    acc_ref[0:R0, :] += jnp.where(valid, x0_ref[...] * cm, 0.0)
    acc_ref[R0:R0 + R1, :] += jnp.where(valid, x1_ref[...] * cm, 0.0)
    acc_ref[R0 + R1:, :] += jnp.where(valid, x2_ref[...] * cm, 0.0)

    @pl.when(b == GRID_B - 1)
    def _():
        nr = R0 + R1 + R2
        rs = jnp.sum(acc_ref[...], axis=1, keepdims=True)
        r3 = lax.broadcasted_iota(jnp.int32, (nr, 3), 0)
        d3 = lax.broadcasted_iota(jnp.int32, (nr, 3), 1)
        bmat = jnp.where(r3 % 3 == d3, rs, 0.0)
        rr = lax.broadcasted_iota(jnp.int32, (3, nr), 1)
        gid = (rr >= R0).astype(jnp.int32) + (rr >= R0 + R1).astype(jnp.int32)
        gg = lax.broadcasted_iota(jnp.int32, (3, nr), 0)
        amat = (gid == gg).astype(jnp.float32)
        out_ref[...] = jnp.dot(amat, bmat, preferred_element_type=jnp.float32)


def _tc_reduce(cnt, x0, x1, x2):
    return pl.pallas_call(
        _tc_body,
        grid=(GRID_B,),
        in_specs=[
            pl.BlockSpec((1, VB), lambda b: (0, b)),
            pl.BlockSpec((R0, VB), lambda b: (0, b)),
            pl.BlockSpec((R1, VB), lambda b: (0, b)),
            pl.BlockSpec((R2, VB), lambda b: (0, b)),
        ],
        out_specs=pl.BlockSpec((3, 3), lambda b: (0, 0)),
        out_shape=jax.ShapeDtypeStruct((3, 3), jnp.float32),
        scratch_shapes=[pltpu.VMEM((R0 + R1 + R2, VB), jnp.float32)],
    )(cnt, x0, x1, x2)


def kernel(e_input, tables0, tables1, tables2):
    idx = e_input.astype(jnp.int32)
    cnt = _sc_histogram(idx).reshape(1, VOCAB)
    x0 = jnp.transpose(tables0, (0, 2, 1)).reshape(R0, VOCAB)
    x1 = jnp.transpose(tables1, (0, 2, 1)).reshape(R1, VOCAB)
    x2 = jnp.transpose(tables2, (0, 2, 1)).reshape(R2, VOCAB)
    return _tc_reduce(cnt, x0, x1, x2)


# speedup vs baseline: 47.1359x; 1.0001x over previous
import functools

import jax
import jax.numpy as jnp
from jax import lax
from jax.experimental import pallas as pl
from jax.experimental.pallas import tpu as pltpu
from jax.experimental.pallas import tpu_sc as plsc

VOCAB = 100000
BATCH = 16384
LANES = 16
NW = 32
B_PER_W = BATCH // NW
N_CHUNK = B_PER_W // 128
V_SH = 100352
Z_PER_S = V_SH // 16

VB = 4096
GRID_B = 25
R0, R1, R2 = 15, 30, 18

_mesh = plsc.VectorSubcoreMesh(core_axis_name="c", subcore_axis_name="s")


@functools.partial(
    pl.kernel,
    mesh=_mesh,
    out_type=jax.ShapeDtypeStruct((2 * VOCAB,), jnp.float32),
    scratch_types=[
        pltpu.VMEM((N_CHUNK, 128), jnp.int32),
        pltpu.VMEM((128,), jnp.float32),
        pltpu.VMEM((Z_PER_S,), jnp.float32),
        pltpu.VMEM_SHARED((V_SH,), jnp.float32),
    ],
    compiler_params=pltpu.CompilerParams(needs_layout_passes=False),
)
def _sc_histogram(idx_hbm, out_hbm, idx_v, ones_v, zero_v, cnt_sh):
    cid = lax.axis_index("c")
    sid = lax.axis_index("s")
    wid = sid * 2 + cid

    pltpu.sync_copy(idx_hbm.at[wid], idx_v)

    ones16 = jnp.ones((LANES,), jnp.float32)
    zeros16 = jnp.zeros((LANES,), jnp.float32)
    for i in range(128 // LANES):
        ones_v[pl.ds(i * LANES, LANES)] = ones16

    def _zero(i, carry):
        zero_v[pl.ds(i * LANES, LANES)] = zeros16
        return carry

    lax.fori_loop(0, Z_PER_S // LANES, _zero, 0)

    pltpu.sync_copy(zero_v, cnt_sh.at[pl.ds(sid * Z_PER_S, Z_PER_S)])
    plsc.subcore_barrier()

    for j in range(N_CHUNK):
        pltpu.sync_copy(ones_v, cnt_sh.at[idx_v.at[j]], add=True)
    plsc.subcore_barrier()

    n_tail = VOCAB - 15 * Z_PER_S

    pltpu.sync_copy(cnt_sh.at[pl.ds(sid * Z_PER_S, Z_PER_S)], zero_v)

    @pl.when(sid < 15)
    def _():
        pltpu.sync_copy(zero_v,
                        out_hbm.at[pl.ds(cid * VOCAB + sid * Z_PER_S, Z_PER_S)])

    @pl.when(sid == 15)
    def _():
        pltpu.sync_copy(zero_v.at[pl.ds(0, n_tail)],
                        out_hbm.at[pl.ds(cid * VOCAB + 15 * Z_PER_S, n_tail)])


def _tc_body(cnt_ref, x0_ref, x1_ref, x2_ref, out_ref, acc_ref):
    b = pl.program_id(0)

    @pl.when(b == 0)
    def _():
        acc_ref[...] = jnp.zeros((R0 + R1 + R2, VB), jnp.float32)

    v = b * VB + lax.broadcasted_iota(jnp.int32, (1, VB), 1)
    valid = v < VOCAB
    cm = cnt_ref[0:1, :] + cnt_ref[1:2, :]
    acc_ref[0:R0, :] += jnp.where(valid, x0_ref[...] * cm, 0.0)
    acc_ref[R0:R0 + R1, :] += jnp.where(valid, x1_ref[...] * cm, 0.0)
    acc_ref[R0 + R1:, :] += jnp.where(valid, x2_ref[...] * cm, 0.0)

    @pl.when(b == GRID_B - 1)
    def _():
        nr = R0 + R1 + R2
        rs = jnp.sum(acc_ref[...], axis=1, keepdims=True)
        r3 = lax.broadcasted_iota(jnp.int32, (nr, 3), 0)
        d3 = lax.broadcasted_iota(jnp.int32, (nr, 3), 1)
        bmat = jnp.where(r3 % 3 == d3, rs, 0.0)
        rr = lax.broadcasted_iota(jnp.int32, (3, nr), 1)
        gid = (rr >= R0).astype(jnp.int32) + (rr >= R0 + R1).astype(jnp.int32)
        gg = lax.broadcasted_iota(jnp.int32, (3, nr), 0)
        amat = (gid == gg).astype(jnp.float32)
        out_ref[...] = jnp.dot(amat, bmat, preferred_element_type=jnp.float32)


def _tc_reduce(cnt, x0, x1, x2):
    return pl.pallas_call(
        _tc_body,
        grid=(GRID_B,),
        in_specs=[
            pl.BlockSpec((2, VB), lambda b: (0, b)),
            pl.BlockSpec((R0, VB), lambda b: (0, b)),
            pl.BlockSpec((R1, VB), lambda b: (0, b)),
            pl.BlockSpec((R2, VB), lambda b: (0, b)),
        ],
        out_specs=pl.BlockSpec((3, 3), lambda b: (0, 0)),
        out_shape=jax.ShapeDtypeStruct((3, 3), jnp.float32),
        scratch_shapes=[pltpu.VMEM((R0 + R1 + R2, VB), jnp.float32)],
    )(cnt, x0, x1, x2)


def kernel(e_input, tables0, tables1, tables2):
    idx = e_input.astype(jnp.int32).reshape(NW, N_CHUNK, 128)
    cnt = _sc_histogram(idx).reshape(2, VOCAB)
    x0 = jnp.transpose(tables0, (0, 2, 1)).reshape(R0, VOCAB)
    x1 = jnp.transpose(tables1, (0, 2, 1)).reshape(R1, VOCAB)
    x2 = jnp.transpose(tables2, (0, 2, 1)).reshape(R2, VOCAB)
    return _tc_reduce(cnt, x0, x1, x2)


# speedup vs baseline: 47.9194x; 1.0166x over previous
import functools

import jax
import jax.numpy as jnp
from jax import lax
from jax.experimental import pallas as pl
from jax.experimental.pallas import tpu as pltpu
from jax.experimental.pallas import tpu_sc as plsc

VOCAB = 100000
BATCH = 16384
LANES = 16
NW = 32
B_PER_W = BATCH // NW
N_CHUNK = B_PER_W // 128
V_SH = 100352
Z_PER_S = V_SH // 16

VB = 4096
GRID_B = 25
R0, R1, R2 = 15, 30, 18

_mesh = plsc.VectorSubcoreMesh(core_axis_name="c", subcore_axis_name="s")


@functools.partial(
    pl.kernel,
    mesh=_mesh,
    out_type=jax.ShapeDtypeStruct((2 * VOCAB,), jnp.float32),
    scratch_types=[
        pltpu.VMEM((N_CHUNK, 128), jnp.int32),
        pltpu.VMEM((128,), jnp.float32),
        pltpu.VMEM((Z_PER_S,), jnp.float32),
        pltpu.VMEM_SHARED((V_SH,), jnp.float32),
    ],
    compiler_params=pltpu.CompilerParams(needs_layout_passes=False),
)
def _sc_histogram(idx_hbm, out_hbm, idx_v, ones_v, zero_v, cnt_sh):
    cid = lax.axis_index("c")
    sid = lax.axis_index("s")
    wid = sid * 2 + cid

    pltpu.sync_copy(idx_hbm.at[wid], idx_v)

    ones16 = jnp.ones((LANES,), jnp.float32)
    zeros16 = jnp.zeros((LANES,), jnp.float32)
    for i in range(128 // LANES):
        ones_v[pl.ds(i * LANES, LANES)] = ones16

    def _zero(i, carry):
        zero_v[pl.ds(i * LANES, LANES)] = zeros16
        return carry

    lax.fori_loop(0, Z_PER_S // LANES, _zero, 0)

    pltpu.sync_copy(zero_v, cnt_sh.at[pl.ds(sid * Z_PER_S, Z_PER_S)])
    plsc.subcore_barrier()

    for j in range(N_CHUNK):
        pltpu.sync_copy(ones_v, cnt_sh.at[idx_v.at[j]], add=True)
    plsc.subcore_barrier()

    n_tail = VOCAB - 15 * Z_PER_S

    pltpu.sync_copy(cnt_sh.at[pl.ds(sid * Z_PER_S, Z_PER_S)], zero_v)

    @pl.when(sid < 15)
    def _():
        pltpu.sync_copy(zero_v,
                        out_hbm.at[pl.ds(cid * VOCAB + sid * Z_PER_S, Z_PER_S)])

    @pl.when(sid == 15)
    def _():
        pltpu.sync_copy(zero_v.at[pl.ds(0, n_tail)],
                        out_hbm.at[pl.ds(cid * VOCAB + 15 * Z_PER_S, n_tail)])


def _fold128(p):
    s = p[:, 0:128]
    for k in range(1, VB // 128):
        s = s + p[:, k * 128:(k + 1) * 128]
    return s


def _tc_body(cnt_ref, x0_ref, x1_ref, x2_ref, out_ref, acc_ref):
    b = pl.program_id(0)

    @pl.when(b == 0)
    def _():
        acc_ref[...] = jnp.zeros((R0 + R1 + R2, 128), jnp.float32)

    cm = cnt_ref[0:1, :] + cnt_ref[1:2, :]

    def _accum(masked):
        if masked:
            v = b * VB + lax.broadcasted_iota(jnp.int32, (1, VB), 1)
            valid = v < VOCAB
            cmm = jnp.where(valid, cm, 0.0)
            mk = lambda x: jnp.where(valid, x.astype(jnp.float32) * cmm, 0.0)
        else:
            mk = lambda x: x.astype(jnp.float32) * cm
        acc_ref[0:R0, :] += _fold128(mk(x0_ref[...]))
        acc_ref[R0:R0 + R1, :] += _fold128(mk(x1_ref[...]))
        acc_ref[R0 + R1:, :] += _fold128(mk(x2_ref[...]))

    @pl.when(b < GRID_B - 1)
    def _():
        _accum(False)

    @pl.when(b == GRID_B - 1)
    def _():
        _accum(True)
        nr = R0 + R1 + R2
        rs = jnp.sum(acc_ref[...], axis=1, keepdims=True)
        r3 = lax.broadcasted_iota(jnp.int32, (nr, 3), 0)
        d3 = lax.broadcasted_iota(jnp.int32, (nr, 3), 1)
        bmat = jnp.where(r3 % 3 == d3, rs, 0.0)
        rr = lax.broadcasted_iota(jnp.int32, (3, nr), 1)
        gid = (rr >= R0).astype(jnp.int32) + (rr >= R0 + R1).astype(jnp.int32)
        gg = lax.broadcasted_iota(jnp.int32, (3, nr), 0)
        amat = (gid == gg).astype(jnp.float32)
        out_ref[...] = jnp.dot(amat, bmat, preferred_element_type=jnp.float32)


def _tc_reduce(cnt, x0, x1, x2):
    return pl.pallas_call(
        _tc_body,
        grid=(GRID_B,),
        in_specs=[
            pl.BlockSpec((2, VB), lambda b: (0, b)),
            pl.BlockSpec((R0, VB), lambda b: (0, b)),
            pl.BlockSpec((R1, VB), lambda b: (0, b)),
            pl.BlockSpec((R2, VB), lambda b: (0, b)),
        ],
        out_specs=pl.BlockSpec((3, 3), lambda b: (0, 0)),
        out_shape=jax.ShapeDtypeStruct((3, 3), jnp.float32),
        scratch_shapes=[pltpu.VMEM((R0 + R1 + R2, 128), jnp.float32)],
    )(cnt, x0, x1, x2)


def kernel(e_input, tables0, tables1, tables2):
    idx = e_input.astype(jnp.int32).reshape(NW, N_CHUNK, 128)
    cnt = _sc_histogram(idx).reshape(2, VOCAB)
    x0 = jnp.transpose(tables0, (0, 2, 1)).reshape(R0, VOCAB).astype(jnp.bfloat16)
    x1 = jnp.transpose(tables1, (0, 2, 1)).reshape(R1, VOCAB).astype(jnp.bfloat16)
    x2 = jnp.transpose(tables2, (0, 2, 1)).reshape(R2, VOCAB).astype(jnp.bfloat16)
    return _tc_reduce(cnt, x0, x1, x2)


# speedup vs baseline: 52.6541x; 1.0988x over previous
import functools

import jax
import jax.numpy as jnp
from jax import lax
from jax.experimental import pallas as pl
from jax.experimental.pallas import tpu as pltpu
from jax.experimental.pallas import tpu_sc as plsc

VOCAB = 100000
BATCH = 16384
LANES = 16
NW = 32
B_PER_T = BATCH // 16
N_CHUNK = B_PER_T // 128
V_SH = 102400
Z_PER_S = V_SH // 16
V_SLICE = 3200
V_LAST = 768
V_TAIL = VOCAB - 31 * V_SLICE - V_LAST
NT = (5, 10, 6)
NR = 9

_mesh = plsc.VectorSubcoreMesh(core_axis_name="c", subcore_axis_name="s")


@functools.partial(
    pl.kernel,
    mesh=_mesh,
    out_type=jax.ShapeDtypeStruct((NW * NR * LANES,), jnp.float32),
    scratch_types=[
        pltpu.VMEM((16, N_CHUNK, 128), jnp.int32),
        pltpu.VMEM((128,), jnp.float32),
        pltpu.VMEM((Z_PER_S,), jnp.float32),
        pltpu.VMEM((V_SLICE,), jnp.float32),
        pltpu.VMEM((3, V_SLICE), jnp.float32),
        pltpu.VMEM((NR * LANES,), jnp.float32),
        pltpu.VMEM((21 * 3 * V_TAIL,), jnp.float32),
        pltpu.VMEM_SHARED((V_SH,), jnp.float32),
    ],
    compiler_params=pltpu.CompilerParams(needs_layout_passes=False),
)
def _sc_fused(idx_hbm, t0_hbm, t1_hbm, t2_hbm, tail_hbm, out_hbm,
              idx_v, ones_v, zero_v, cnt_v, tbuf_v, acc_v, tail_v, cnt_sh):
    cid = lax.axis_index("c")
    sid = lax.axis_index("s")
    wid = sid * 2 + cid
    vb = wid * V_SLICE

    pltpu.sync_copy(idx_hbm, idx_v)

    ones16 = jnp.ones((LANES,), jnp.float32)
    zeros16 = jnp.zeros((LANES,), jnp.float32)
    for i in range(128 // LANES):
        ones_v[pl.ds(i * LANES, LANES)] = ones16

    def _zero(i, carry):
        zero_v[pl.ds(i * LANES, LANES)] = zeros16
        return carry

    lax.fori_loop(0, Z_PER_S // LANES, _zero, 0)
    pltpu.sync_copy(zero_v, cnt_sh.at[pl.ds(sid * Z_PER_S, Z_PER_S)])
    plsc.subcore_barrier()

    for j in range(N_CHUNK):
        pltpu.sync_copy(ones_v, cnt_sh.at[idx_v.at[sid, j]], add=True)
    plsc.subcore_barrier()

    pltpu.sync_copy(cnt_sh.at[pl.ds(vb, V_SLICE)], cnt_v)

    last = wid == NW - 1
    klen = jnp.where(last, V_LAST // LANES, V_SLICE // LANES)
    pltpu.sync_copy(tail_hbm, tail_v)
    lastf = jnp.where(last, jnp.ones((LANES,), jnp.float32),
                      jnp.zeros((LANES,), jnp.float32))
    ct0 = cnt_v[pl.ds(V_LAST, LANES)] * lastf
    ct1 = cnt_v[pl.ds(V_LAST + LANES, LANES)] * lastf

    accs = []
    tg = 0
    for tref, ntab in ((t0_hbm, NT[0]), (t1_hbm, NT[1]), (t2_hbm, NT[2])):
        a0 = jnp.zeros((LANES,), jnp.float32)
        a1 = jnp.zeros((LANES,), jnp.float32)
        a2 = jnp.zeros((LANES,), jnp.float32)
        for t in range(ntab):
            @pl.when(jnp.logical_not(last))
            def _(t=t, tref=tref):
                pltpu.sync_copy(tref.at[t, :, pl.ds(vb, V_SLICE)], tbuf_v)

            @pl.when(last)
            def _(t=t, tref=tref):
                pltpu.sync_copy(tref.at[t, :, pl.ds(31 * V_SLICE, V_LAST)],
                                tbuf_v.at[:, pl.ds(0, V_LAST)])

            def _mac(k, carry):
                b0, b1, b2 = carry
                o = k * LANES
                c16 = cnt_v[pl.ds(o, LANES)]
                b0 = b0 + tbuf_v[0, pl.ds(o, LANES)] * c16
                b1 = b1 + tbuf_v[1, pl.ds(o, LANES)] * c16
                b2 = b2 + tbuf_v[2, pl.ds(o, LANES)] * c16
                return b0, b1, b2

            a0, a1, a2 = lax.fori_loop(0, klen, _mac, (a0, a1, a2))
            base = tg * 3 * V_TAIL
            a0 = a0 + tail_v[pl.ds(base, LANES)] * ct0
            a0 = a0 + tail_v[pl.ds(base + LANES, LANES)] * ct1
            a1 = a1 + tail_v[pl.ds(base + V_TAIL, LANES)] * ct0
            a1 = a1 + tail_v[pl.ds(base + V_TAIL + LANES, LANES)] * ct1
            a2 = a2 + tail_v[pl.ds(base + 2 * V_TAIL, LANES)] * ct0
            a2 = a2 + tail_v[pl.ds(base + 2 * V_TAIL + LANES, LANES)] * ct1
            tg += 1
        accs += [a0, a1, a2]

    for a in range(NR):
        acc_v[pl.ds(a * LANES, LANES)] = accs[a]
    pltpu.sync_copy(acc_v, out_hbm.at[pl.ds(wid * NR * LANES, NR * LANES)])


def _tc_fold_body(p_ref, out_ref):
    n = NW * NR
    rs = jnp.sum(p_ref[...], axis=1, keepdims=True)
    j3 = lax.broadcasted_iota(jnp.int32, (n, 3), 0) % NR
    d3 = lax.broadcasted_iota(jnp.int32, (n, 3), 1)
    cmat = jnp.where(j3 % 3 == d3, rs, 0.0)
    jj = lax.broadcasted_iota(jnp.int32, (3, n), 1) % NR
    gg = lax.broadcasted_iota(jnp.int32, (3, n), 0)
    amat = (jj // 3 == gg).astype(jnp.float32)
    out_ref[...] = jnp.dot(amat, cmat, preferred_element_type=jnp.float32)


def _tc_fold(partials):
    return pl.pallas_call(
        _tc_fold_body,
        out_shape=jax.ShapeDtypeStruct((3, 3), jnp.float32),
    )(partials)


def kernel(e_input, tables0, tables1, tables2):
    idx = e_input.astype(jnp.int32).reshape(16, N_CHUNK, 128)
    x0 = jnp.transpose(tables0, (0, 2, 1))
    x1 = jnp.transpose(tables1, (0, 2, 1))
    x2 = jnp.transpose(tables2, (0, 2, 1))
    vt = 31 * V_SLICE + V_LAST
    tail = jnp.concatenate([tables0[:, vt:, :], tables1[:, vt:, :],
                            tables2[:, vt:, :]], axis=0)
    tailx = jnp.transpose(tail, (0, 2, 1)).reshape(21 * 3 * V_TAIL)
    partials = _sc_fused(idx, x0, x1, x2, tailx).reshape(NW * NR, LANES)
    return _tc_fold(partials)


# speedup vs baseline: 78.0180x; 1.4817x over previous
import functools

import jax
import jax.numpy as jnp
from jax import lax
from jax.experimental import pallas as pl
from jax.experimental.pallas import tpu as pltpu
from jax.experimental.pallas import tpu_sc as plsc

VOCAB = 100000
BATCH = 16384
LANES = 16
NW = 32
B_PER_T = BATCH // 16
N_CHUNK = B_PER_T // 128
V_SH = 102400
Z_PER_S = V_SH // 16
V_SLICE = 3200
V_LB = 96768
V_PREF = 31 * V_SLICE - V_LB
V_TAIL = VOCAB - V_LB - V_SLICE
NT = (5, 10, 6)
NTOT = sum(NT)
NR = 9
NBUF = 3

_mesh = plsc.VectorSubcoreMesh(core_axis_name="c", subcore_axis_name="s")


@functools.partial(
    pl.kernel,
    mesh=_mesh,
    out_type=jax.ShapeDtypeStruct((NW * NR * LANES,), jnp.float32),
    scratch_types=[
        pltpu.VMEM((16, N_CHUNK, 128), jnp.int32),
        pltpu.VMEM((128,), jnp.float32),
        pltpu.VMEM((Z_PER_S,), jnp.float32),
        pltpu.VMEM((V_SLICE,), jnp.float32),
        pltpu.VMEM((NBUF, 3, V_SLICE), jnp.float32),
        pltpu.VMEM((NR * LANES,), jnp.float32),
        pltpu.VMEM((NTOT * 3 * V_TAIL,), jnp.float32),
        pltpu.VMEM_SHARED((V_SH,), jnp.float32),
        pltpu.SemaphoreType.DMA,
        pltpu.SemaphoreType.DMA,
        pltpu.SemaphoreType.DMA,
    ],
    compiler_params=pltpu.CompilerParams(needs_layout_passes=False),
)
def _sc_fused(idx_hbm, t0_hbm, t1_hbm, t2_hbm, tail_hbm, out_hbm,
              idx_v, ones_v, zero_v, cnt_v, tbuf_v, acc_v, tail_v, cnt_sh,
              sem0, sem1, sem2):
    cid = lax.axis_index("c")
    sid = lax.axis_index("s")
    wid = sid * 2 + cid
    last = wid == NW - 1
    vb = jnp.where(last, V_LB, wid * V_SLICE)
    sems = (sem0, sem1, sem2)
    tabs = ([(t0_hbm, t) for t in range(NT[0])]
            + [(t1_hbm, t) for t in range(NT[1])]
            + [(t2_hbm, t) for t in range(NT[2])])

    def _issue(i):
        tref, t = tabs[i]
        return pltpu.async_copy(tref.at[t, :, pl.ds(vb, V_SLICE)],
                                tbuf_v.at[i % NBUF], sems[i % NBUF])

    handles = {i: _issue(i) for i in range(NBUF)}

    pltpu.sync_copy(idx_hbm, idx_v)
    pltpu.sync_copy(tail_hbm, tail_v)

    ones16 = jnp.ones((LANES,), jnp.float32)
    zeros16 = jnp.zeros((LANES,), jnp.float32)
    for i in range(128 // LANES):
        ones_v[pl.ds(i * LANES, LANES)] = ones16

    def _zero(i, carry):
        zero_v[pl.ds(i * LANES, LANES)] = zeros16
        return carry

    lax.fori_loop(0, Z_PER_S // LANES, _zero, 0)
    pltpu.sync_copy(zero_v, cnt_sh.at[pl.ds(sid * Z_PER_S, Z_PER_S)])
    plsc.subcore_barrier()

    for j in range(N_CHUNK):
        pltpu.sync_copy(ones_v, cnt_sh.at[idx_v.at[sid, j]], add=True)
    plsc.subcore_barrier()

    pltpu.sync_copy(cnt_sh.at[pl.ds(vb, V_SLICE)], cnt_v)

    @pl.when(last)
    def _():
        def _zp(i, carry):
            cnt_v[pl.ds(i * LANES, LANES)] = zeros16
            return carry
        lax.fori_loop(0, V_PREF // LANES, _zp, 0)

    lastf = jnp.where(last, ones16, zeros16)
    ct0 = cnt_v[pl.ds(V_SLICE - 2 * LANES, LANES)] * lastf
    ct1 = cnt_v[pl.ds(V_SLICE - LANES, LANES)] * lastf

    accs = []
    a0 = a1 = a2 = jnp.zeros((LANES,), jnp.float32)
    for i in range(NTOT):
        slot = i % NBUF
        handles[i].wait()

        def _mac(k, carry, slot=slot):
            b0, b1, b2 = carry
            for u in range(4):
                o = (k * 4 + u) * LANES
                c16 = cnt_v[pl.ds(o, LANES)]
                b0 = b0 + tbuf_v[slot, 0, pl.ds(o, LANES)] * c16
                b1 = b1 + tbuf_v[slot, 1, pl.ds(o, LANES)] * c16
                b2 = b2 + tbuf_v[slot, 2, pl.ds(o, LANES)] * c16
            return b0, b1, b2

        r0, r1, r2 = lax.fori_loop(0, V_SLICE // (4 * LANES), _mac,
                                   (jnp.zeros((LANES,), jnp.float32),) * 3)
        base = i * 3 * V_TAIL
        r0 = r0 + tail_v[pl.ds(base, LANES)] * ct0
        r0 = r0 + tail_v[pl.ds(base + LANES, LANES)] * ct1
        r1 = r1 + tail_v[pl.ds(base + V_TAIL, LANES)] * ct0
        r1 = r1 + tail_v[pl.ds(base + V_TAIL + LANES, LANES)] * ct1
        r2 = r2 + tail_v[pl.ds(base + 2 * V_TAIL, LANES)] * ct0
        r2 = r2 + tail_v[pl.ds(base + 2 * V_TAIL + LANES, LANES)] * ct1
        a0, a1, a2 = a0 + r0, a1 + r1, a2 + r2

        if i + NBUF < NTOT:
            handles[i + NBUF] = _issue(i + NBUF)
        if i in (NT[0] - 1, NT[0] + NT[1] - 1, NTOT - 1):
            accs += [a0, a1, a2]
            a0 = a1 = a2 = jnp.zeros((LANES,), jnp.float32)

    for a in range(NR):
        acc_v[pl.ds(a * LANES, LANES)] = accs[a]
    pltpu.sync_copy(acc_v, out_hbm.at[pl.ds(wid * NR * LANES, NR * LANES)])


def _tc_fold_body(p_ref, out_ref):
    n = NW * NR
    rs = jnp.sum(p_ref[...], axis=1, keepdims=True)
    j3 = lax.broadcasted_iota(jnp.int32, (n, 3), 0) % NR
    d3 = lax.broadcasted_iota(jnp.int32, (n, 3), 1)
    cmat = jnp.where(j3 % 3 == d3, rs, 0.0)
    jj = lax.broadcasted_iota(jnp.int32, (3, n), 1) % NR
    gg = lax.broadcasted_iota(jnp.int32, (3, n), 0)
    amat = (jj // 3 == gg).astype(jnp.float32)
    out_ref[...] = jnp.dot(amat, cmat, preferred_element_type=jnp.float32)


def _tc_fold(partials):
    return pl.pallas_call(
        _tc_fold_body,
        out_shape=jax.ShapeDtypeStruct((3, 3), jnp.float32),
    )(partials)


def kernel(e_input, tables0, tables1, tables2):
    idx = e_input.astype(jnp.int32).reshape(16, N_CHUNK, 128)
    x0 = jnp.transpose(tables0, (0, 2, 1))
    x1 = jnp.transpose(tables1, (0, 2, 1))
    x2 = jnp.transpose(tables2, (0, 2, 1))
    vt = V_LB + V_SLICE
    tail = jnp.concatenate([tables0[:, vt:, :], tables1[:, vt:, :],
                            tables2[:, vt:, :]], axis=0)
    tailx = jnp.transpose(tail, (0, 2, 1)).reshape(NTOT * 3 * V_TAIL)
    partials = _sc_fused(idx, x0, x1, x2, tailx).reshape(NW * NR, LANES)
    return _tc_fold(partials)
